# baseline (speedup 1.0000x reference)
import dataclasses
import functools

import jax
import jax.numpy as jnp
from jax import lax
from jax.experimental import pallas as pl
from jax.experimental.pallas import tpu as pltpu
from jax.experimental.pallas import tpu_sc as plsc

_N = 10000
_E = 320000
_D = 128
_H = 128
_C = 64
_K = 8

_NPAD = 10240
_NC = 2
_NS = 16
_CHUNK = 128
_CPW = 80
_EPW = _CHUNK * _CPW
_EPAD = _EPW * _NC * _NS
_IBLK = 16

_B1 = 1000
_B3 = 1000


def _deg_partials(col_pad, zeros1):
    mesh = plsc.VectorSubcoreMesh(core_axis_name="c", subcore_axis_name="s")
    cp = pltpu.CompilerParams()
    if "needs_layout_passes" in pltpu.CompilerParams.__dataclass_fields__:
        cp = dataclasses.replace(cp, needs_layout_passes=False)

    @functools.partial(
        pl.kernel, mesh=mesh, compiler_params=cp,
        out_type=jax.ShapeDtypeStruct((_NC * _NS, _NPAD), jnp.float32),
        scratch_types=[
            pltpu.VMEM((_CPW, _CHUNK), jnp.int32),
            pltpu.VMEM((_NPAD,), jnp.float32),
        ],
    )
    def k(col_hbm, z1_hbm, out_hbm, cidx, hist):
        cid = lax.axis_index("c")
        sid = lax.axis_index("s")
        wid = cid * _NS + sid
        pltpu.sync_copy(z1_hbm, hist)
        pltpu.sync_copy(col_hbm.at[pl.ds(wid * _CPW, _CPW)], cidx)
        ones16 = jnp.full((16,), 1.0, jnp.float32)

        @pl.loop(0, _CPW)
        def _(c):
            @pl.loop(0, _CHUNK // 16)
            def _(j):
                idxv = cidx[c, pl.ds(j * 16, 16)]
                plsc.addupdate_scatter(hist, [idxv], ones16)

        pltpu.sync_copy(hist, out_hbm.at[wid])

    return k(col_pad, zeros1)


def _gcn_partials(hs, row_pad, col_pad, zeros_acc):
    mesh = plsc.VectorSubcoreMesh(core_axis_name="c", subcore_axis_name="s")

    @functools.partial(
        pl.kernel, mesh=mesh,
        out_type=jax.ShapeDtypeStruct((_NC, _NPAD, _H), jnp.float32),
        scratch_types=[
            pltpu.VMEM((_IBLK, _CHUNK), jnp.int32),
            pltpu.VMEM((_IBLK, _CHUNK), jnp.int32),
            pltpu.VMEM((_CHUNK, _H), jnp.float32),
            pltpu.VMEM((_CHUNK, _H), jnp.float32),
            pltpu.VMEM_SHARED((_NPAD, _H), jnp.float32),
            pltpu.SemaphoreType.DMA,
            pltpu.SemaphoreType.DMA,
        ],
    )
    def k(hs_hbm, row_hbm, col_hbm, z_hbm, out_hbm, ridx, cidx, rows0, rows1,
          acc, sem0, sem1):
        cid = lax.axis_index("c")
        sid = lax.axis_index("s")
        rows_per = _NPAD // _NS
        pltpu.sync_copy(z_hbm.at[pl.ds(sid * rows_per, rows_per)],
                        acc.at[pl.ds(sid * rows_per, rows_per)])
        base_c = (cid * _NS + sid) * _CPW
        plsc.subcore_barrier()

        @pl.loop(0, _CPW, step=_IBLK)
        def _(b):
            pltpu.sync_copy(row_hbm.at[pl.ds(base_c + b, _IBLK)], ridx)
            pltpu.sync_copy(col_hbm.at[pl.ds(base_c + b, _IBLK)], cidx)
            pltpu.async_copy(hs_hbm.at[ridx.at[0]], rows0, sem0)

            @pl.loop(0, _IBLK, step=2)
            def _(c):
                pltpu.async_copy(hs_hbm.at[ridx.at[c + 1]], rows1, sem1)
                pltpu.make_async_copy(hs_hbm.at[ridx.at[c]], rows0,
                                      sem0).wait()
                pltpu.sync_copy(rows0, acc.at[cidx.at[c]], add=True)

                @pl.when(c + 2 < _IBLK)
                def _():
                    pltpu.async_copy(hs_hbm.at[ridx.at[c + 2]], rows0, sem0)

                pltpu.make_async_copy(hs_hbm.at[ridx.at[c + 1]], rows1,
                                      sem1).wait()
                pltpu.sync_copy(rows1, acc.at[cidx.at[c + 1]], add=True)

        plsc.subcore_barrier()
        pltpu.sync_copy(acc.at[pl.ds(sid * rows_per, rows_per)],
                        out_hbm.at[cid, pl.ds(sid * rows_per, rows_per)])

    return k(hs, row_pad, col_pad, zeros_acc)


def _k1_body(x_ref, Win_ref, bin_ref, Wq_ref, bq_ref, Wk_ref, bk_ref,
             Wv_ref, bv_ref,
             h_ref, q_ref, v_ref, ssqq_ref, ssqk_ref, kt1_ref, ktv_ref):
    i = pl.program_id(0)
    xb = x_ref[...]
    h = jnp.maximum(jnp.dot(xb, Win_ref[...],
                            preferred_element_type=jnp.float32) + bin_ref[...], 0.0)
    q = jnp.dot(h, Wq_ref[...],
                preferred_element_type=jnp.float32) + bq_ref[...]
    kk = jnp.dot(h, Wk_ref[...],
                 preferred_element_type=jnp.float32) + bk_ref[...]
    v = jnp.dot(h, Wv_ref[...],
                preferred_element_type=jnp.float32) + bv_ref[...]
    h_ref[...] = h
    q_ref[...] = q
    v_ref[...] = v
    pssqq = jnp.sum(q * q, axis=(0, 1), keepdims=True)
    pssqk = jnp.sum(kk * kk, axis=(0, 1), keepdims=True)
    pkt1 = jnp.sum(kk, axis=0, keepdims=True)
    pktv = lax.dot_general(kk, v, (((0,), (0,)), ((), ())),
                           preferred_element_type=jnp.float32)

    @pl.when(i == 0)
    def _():
        ssqq_ref[...] = pssqq
        ssqk_ref[...] = pssqk
        kt1_ref[...] = pkt1
        ktv_ref[...] = pktv

    @pl.when(i > 0)
    def _():
        ssqq_ref[...] += pssqq
        ssqk_ref[...] += pssqk
        kt1_ref[...] += pkt1
        ktv_ref[...] += pktv


def _k1(x, W_in, b_in, Wq, bq, Wk, bk, Wv, bv):
    nb = _N // _B1
    row_spec = pl.BlockSpec((_B1, _H), lambda i: (i, 0))
    w_spec = pl.BlockSpec((_D, _H), lambda i: (0, 0))
    b_spec = pl.BlockSpec((1, _H), lambda i: (0, 0))
    return pl.pallas_call(
        _k1_body,
        grid=(nb,),
        in_specs=[row_spec, w_spec, b_spec, w_spec, b_spec, w_spec, b_spec,
                  w_spec, b_spec],
        out_specs=[row_spec, row_spec, row_spec,
                   pl.BlockSpec((1, 1), lambda i: (0, 0)),
                   pl.BlockSpec((1, 1), lambda i: (0, 0)),
                   pl.BlockSpec((1, _H), lambda i: (0, 0)),
                   pl.BlockSpec((_H, _H), lambda i: (0, 0))],
        out_shape=[jax.ShapeDtypeStruct((_N, _H), jnp.float32),
                   jax.ShapeDtypeStruct((_N, _H), jnp.float32),
                   jax.ShapeDtypeStruct((_N, _H), jnp.float32),
                   jax.ShapeDtypeStruct((1, 1), jnp.float32),
                   jax.ShapeDtypeStruct((1, 1), jnp.float32),
                   jax.ShapeDtypeStruct((1, _H), jnp.float32),
                   jax.ShapeDtypeStruct((_H, _H), jnp.float32)],
    )(x, W_in, b_in.reshape(1, _H), Wq, bq.reshape(1, _H),
      Wk, bk.reshape(1, _H), Wv, bv.reshape(1, _H))


def _k2_body(q_ref, v_ref, h_ref, deg_ref, ssqq_ref, ssqk_ref, kt1_ref,
             ktv_ref, Wg_ref, bg_ref,
             glog_ref, s_ref, hs_ref):
    qn = jnp.maximum(jnp.sqrt(ssqq_ref[...]), 1e-12)
    kn = jnp.maximum(jnp.sqrt(ssqk_ref[...]), 1e-12)
    sc = 1.0 / (qn * kn * jnp.float32(_N))
    q = q_ref[...]
    v = v_ref[...]
    an = 1.0 + jnp.sum(q * kt1_ref[...], axis=1, keepdims=True) * sc
    grep = (v + jnp.dot(q, ktv_ref[...],
                        preferred_element_type=jnp.float32) * sc)
    grep = grep / jnp.maximum(an, 1e-12)
    glog_ref[...] = jnp.dot(grep, Wg_ref[...],
                            preferred_element_type=jnp.float32) + bg_ref[...]
    deg = deg_ref[...]
    s = jnp.where(deg > 0, 1.0 / jnp.sqrt(jnp.maximum(deg, 1.0)), 0.0)
    s_ref[...] = s
    hs_ref[...] = s * h_ref[...]


def _k2(q, v, h, deg, ssqq, ssqk, kt1, ktv, Wg, bg):
    nb = _N // _B1
    row_spec = pl.BlockSpec((_B1, _H), lambda i: (i, 0))
    return pl.pallas_call(
        _k2_body,
        grid=(nb,),
        in_specs=[row_spec, row_spec, row_spec,
                  pl.BlockSpec((_B1, 1), lambda i: (i, 0)),
                  pl.BlockSpec((1, 1), lambda i: (0, 0)),
                  pl.BlockSpec((1, 1), lambda i: (0, 0)),
                  pl.BlockSpec((1, _H), lambda i: (0, 0)),
                  pl.BlockSpec((_H, _H), lambda i: (0, 0)),
                  pl.BlockSpec((_H, _K), lambda i: (0, 0)),
                  pl.BlockSpec((1, _K), lambda i: (0, 0))],
        out_specs=[pl.BlockSpec((_B1, _K), lambda i: (i, 0)),
                   pl.BlockSpec((_B1, 1), lambda i: (i, 0)),
                   row_spec],
        out_shape=[jax.ShapeDtypeStruct((_N, _K), jnp.float32),
                   jax.ShapeDtypeStruct((_N, 1), jnp.float32),
                   jax.ShapeDtypeStruct((_N, _H), jnp.float32)],
    )(q, v, h, deg, ssqq, ssqk, kt1, ktv, Wg, bg.reshape(1, _K))


def _k3_mid_body(h_ref, p_ref, s_ref, glog_ref, Wenv_ref, benv_ref,
                 Wcg_ref, Wch_ref, h_out_ref, hs_out_ref):
    h = h_ref[...]
    s = s_ref[...]
    p = p_ref[...]
    gcn = s * (p[0] + p[1])
    logits = jnp.dot(h, Wenv_ref[...],
                     preferred_element_type=jnp.float32) + benv_ref[...] + glog_ref[...]
    m = jnp.max(logits, axis=1, keepdims=True)
    e = jnp.exp(logits - m)
    w = e / jnp.sum(e, axis=1, keepdims=True)
    big = (jnp.dot(gcn, Wcg_ref[...], preferred_element_type=jnp.float32)
           + jnp.dot(h, Wch_ref[...], preferred_element_type=jnp.float32))
    acc = h
    for kk in range(_K):
        acc = acc + w[:, kk:kk + 1] * big[:, kk * _H:(kk + 1) * _H]
    hn = jnp.maximum(acc, 0.0)
    h_out_ref[...] = hn
    hs_out_ref[...] = s * hn


def _k3_last_body(h_ref, p_ref, s_ref, glog_ref, Wenv_ref, benv_ref,
                  Wcg_ref, Wch_ref, Wout_ref, bout_ref, y_ref):
    h = h_ref[...]
    s = s_ref[...]
    p = p_ref[...]
    gcn = s * (p[0] + p[1])
    logits = jnp.dot(h, Wenv_ref[...],
                     preferred_element_type=jnp.float32) + benv_ref[...] + glog_ref[...]
    m = jnp.max(logits, axis=1, keepdims=True)
    e = jnp.exp(logits - m)
    w = e / jnp.sum(e, axis=1, keepdims=True)
    big = (jnp.dot(gcn, Wcg_ref[...], preferred_element_type=jnp.float32)
           + jnp.dot(h, Wch_ref[...], preferred_element_type=jnp.float32))
    acc = h
    for kk in range(_K):
        acc = acc + w[:, kk:kk + 1] * big[:, kk * _H:(kk + 1) * _H]
    hn = jnp.maximum(acc, 0.0)
    y_ref[...] = jnp.dot(hn, Wout_ref[...],
                         preferred_element_type=jnp.float32) + bout_ref[...]


def _k3_specs():
    row_spec = pl.BlockSpec((_B3, _H), lambda i: (i, 0))
    return [row_spec,
            pl.BlockSpec((2, _B3, _H), lambda i: (0, i, 0)),
            pl.BlockSpec((_B3, 1), lambda i: (i, 0)),
            pl.BlockSpec((_B3, _K), lambda i: (i, 0)),
            pl.BlockSpec((_H, _K), lambda i: (0, 0)),
            pl.BlockSpec((1, _K), lambda i: (0, 0)),
            pl.BlockSpec((_H, _K * _H), lambda i: (0, 0)),
            pl.BlockSpec((_H, _K * _H), lambda i: (0, 0))]


def _k3_mid(h, p, s, glog, Wenv, benv, Wcg, Wch):
    nb = _N // _B3
    row_spec = pl.BlockSpec((_B3, _H), lambda i: (i, 0))
    return pl.pallas_call(
        _k3_mid_body,
        grid=(nb,),
        in_specs=_k3_specs(),
        out_specs=[row_spec, row_spec],
        out_shape=[jax.ShapeDtypeStruct((_N, _H), jnp.float32),
                   jax.ShapeDtypeStruct((_N, _H), jnp.float32)],
    )(h, p, s, glog, Wenv, benv.reshape(1, _K), Wcg, Wch)


def _k3_last(h, p, s, glog, Wenv, benv, Wcg, Wch, W_out, b_out):
    nb = _N // _B3
    return pl.pallas_call(
        _k3_last_body,
        grid=(nb,),
        in_specs=_k3_specs() + [pl.BlockSpec((_H, _C), lambda i: (0, 0)),
                                pl.BlockSpec((1, _C), lambda i: (0, 0))],
        out_specs=[pl.BlockSpec((_B3, _C), lambda i: (i, 0))],
        out_shape=[jax.ShapeDtypeStruct((_N, _C), jnp.float32)],
    )(h, p, s, glog, Wenv, benv.reshape(1, _K), Wcg, Wch, W_out,
      b_out.reshape(1, _C))[0]


def kernel(x, edge_index, W_in, b_in, Wq, bq, Wk, bk, Wv, bv, Wg, bg,
           W_env_enc, b_env_enc, W_conv, W_out, b_out):
    row = edge_index[0]
    col = edge_index[1]
    npad_e = _EPAD - _E
    row_pad = jnp.concatenate(
        [row, jnp.zeros((npad_e,), jnp.int32)]).reshape(_EPAD // _CHUNK, _CHUNK)
    col_pad = jnp.concatenate(
        [col, jnp.full((npad_e,), _NPAD - 1, jnp.int32)]).reshape(
            _EPAD // _CHUNK, _CHUNK)
    zeros_acc = jnp.zeros((_NPAD, _H), jnp.float32)

    deg2 = _deg_partials(col_pad, jnp.zeros((_NPAD,), jnp.float32))
    deg = jnp.sum(deg2, axis=0)[:_N].reshape(_N, 1)

    h, q, v, ssqq, ssqk, kt1, ktv = _k1(x, W_in, b_in, Wq, bq, Wk, bk,
                                        Wv, bv)
    glog, s, hs = _k2(q, v, h, deg, ssqq, ssqk, kt1, ktv, Wg, bg)

    Wcg0 = W_conv[0][:, :_H, :].transpose(1, 0, 2).reshape(_H, _K * _H)
    Wch0 = W_conv[0][:, _H:, :].transpose(1, 0, 2).reshape(_H, _K * _H)
    Wcg1 = W_conv[1][:, :_H, :].transpose(1, 0, 2).reshape(_H, _K * _H)
    Wch1 = W_conv[1][:, _H:, :].transpose(1, 0, 2).reshape(_H, _K * _H)

    p0 = _gcn_partials(hs, row_pad, col_pad, zeros_acc)
    h, hs = _k3_mid(h, p0, s, glog, W_env_enc[0], b_env_enc[0], Wcg0, Wch0)

    p1 = _gcn_partials(hs, row_pad, col_pad, zeros_acc)
    y = _k3_last(h, p1, s, glog, W_env_enc[1], b_env_enc[1], Wcg1, Wch1,
                 W_out, b_out)
    return y


# speedup vs baseline: 1.0145x; 1.0145x over previous
import dataclasses
import functools

import jax
import jax.numpy as jnp
from jax import lax
from jax.experimental import pallas as pl
from jax.experimental.pallas import tpu as pltpu
from jax.experimental.pallas import tpu_sc as plsc

_N = 10000
_E = 320000
_D = 128
_H = 128
_C = 64
_K = 8

_NPAD = 10240
_NC = 2
_NS = 16
_CHUNK = 128
_CPW = 80
_EPW = _CHUNK * _CPW
_EPAD = _EPW * _NC * _NS
_IBLK = 16

_B1 = 1024
_B3 = 512


def _deg_partials(col_pad, zeros1):
    mesh = plsc.VectorSubcoreMesh(core_axis_name="c", subcore_axis_name="s")
    cp = pltpu.CompilerParams()
    if "needs_layout_passes" in pltpu.CompilerParams.__dataclass_fields__:
        cp = dataclasses.replace(cp, needs_layout_passes=False)

    @functools.partial(
        pl.kernel, mesh=mesh, compiler_params=cp,
        out_type=jax.ShapeDtypeStruct((_NC * _NS, _NPAD), jnp.float32),
        scratch_types=[
            pltpu.VMEM((_CPW, _CHUNK), jnp.int32),
            pltpu.VMEM((_NPAD,), jnp.float32),
        ],
    )
    def k(col_hbm, z1_hbm, out_hbm, cidx, hist):
        cid = lax.axis_index("c")
        sid = lax.axis_index("s")
        wid = cid * _NS + sid
        pltpu.sync_copy(z1_hbm, hist)
        pltpu.sync_copy(col_hbm.at[pl.ds(wid * _CPW, _CPW)], cidx)
        ones16 = jnp.full((16,), 1.0, jnp.float32)

        @pl.loop(0, _CPW)
        def _(c):
            @pl.loop(0, _CHUNK // 16)
            def _(j):
                idxv = cidx[c, pl.ds(j * 16, 16)]
                plsc.addupdate_scatter(hist, [idxv], ones16)

        pltpu.sync_copy(hist, out_hbm.at[wid])

    return k(col_pad, zeros1)


def _gcn_partials(hs, row_pad, col_pad, zeros_acc):
    mesh = plsc.VectorSubcoreMesh(core_axis_name="c", subcore_axis_name="s")

    @functools.partial(
        pl.kernel, mesh=mesh,
        out_type=jax.ShapeDtypeStruct((_NC, _NPAD, _H), jnp.float32),
        scratch_types=[
            pltpu.VMEM((_IBLK, _CHUNK), jnp.int32),
            pltpu.VMEM((_IBLK, _CHUNK), jnp.int32),
            pltpu.VMEM((_CHUNK, _H), jnp.float32),
            pltpu.VMEM((_CHUNK, _H), jnp.float32),
            pltpu.VMEM_SHARED((_NPAD, _H), jnp.float32),
            pltpu.SemaphoreType.DMA,
            pltpu.SemaphoreType.DMA,
        ],
    )
    def k(hs_hbm, row_hbm, col_hbm, z_hbm, out_hbm, ridx, cidx, rows0, rows1,
          acc, sem0, sem1):
        cid = lax.axis_index("c")
        sid = lax.axis_index("s")
        rows_per = _NPAD // _NS
        pltpu.sync_copy(z_hbm.at[pl.ds(sid * rows_per, rows_per)],
                        acc.at[pl.ds(sid * rows_per, rows_per)])
        base_c = (cid * _NS + sid) * _CPW
        plsc.subcore_barrier()

        @pl.loop(0, _CPW, step=_IBLK)
        def _(b):
            pltpu.sync_copy(row_hbm.at[pl.ds(base_c + b, _IBLK)], ridx)
            pltpu.sync_copy(col_hbm.at[pl.ds(base_c + b, _IBLK)], cidx)
            pltpu.async_copy(hs_hbm.at[ridx.at[0]], rows0, sem0)

            @pl.loop(0, _IBLK, step=2)
            def _(c):
                pltpu.async_copy(hs_hbm.at[ridx.at[c + 1]], rows1, sem1)
                pltpu.make_async_copy(hs_hbm.at[ridx.at[c]], rows0,
                                      sem0).wait()
                pltpu.sync_copy(rows0, acc.at[cidx.at[c]], add=True)

                @pl.when(c + 2 < _IBLK)
                def _():
                    pltpu.async_copy(hs_hbm.at[ridx.at[c + 2]], rows0, sem0)

                pltpu.make_async_copy(hs_hbm.at[ridx.at[c + 1]], rows1,
                                      sem1).wait()
                pltpu.sync_copy(rows1, acc.at[cidx.at[c + 1]], add=True)

        plsc.subcore_barrier()
        pltpu.sync_copy(acc.at[pl.ds(sid * rows_per, rows_per)],
                        out_hbm.at[cid, pl.ds(sid * rows_per, rows_per)])

    return k(hs, row_pad, col_pad, zeros_acc)


def _k1_body(x_ref, Win_ref, bin_ref, Wq_ref, bq_ref, Wk_ref, bk_ref,
             Wv_ref, bv_ref,
             h_ref, q_ref, v_ref, ssqq_ref, ssqk_ref, kt1_ref, ktv_ref):
    i = pl.program_id(0)
    xb = x_ref[...]
    h = jnp.maximum(jnp.dot(xb, Win_ref[...],
                            preferred_element_type=jnp.float32) + bin_ref[...], 0.0)
    ridx = lax.broadcasted_iota(jnp.int32, (_B1, 1), 0) + i * _B1
    mask = ridx < _N
    h = jnp.where(mask, h, 0.0)
    q = jnp.where(mask, jnp.dot(h, Wq_ref[...],
                                preferred_element_type=jnp.float32) + bq_ref[...], 0.0)
    kk = jnp.where(mask, jnp.dot(h, Wk_ref[...],
                                 preferred_element_type=jnp.float32) + bk_ref[...], 0.0)
    v = jnp.where(mask, jnp.dot(h, Wv_ref[...],
                                preferred_element_type=jnp.float32) + bv_ref[...], 0.0)
    h_ref[...] = h
    q_ref[...] = q
    v_ref[...] = v
    pssqq = jnp.sum(q * q, axis=(0, 1), keepdims=True)
    pssqk = jnp.sum(kk * kk, axis=(0, 1), keepdims=True)
    pkt1 = jnp.sum(kk, axis=0, keepdims=True)
    pktv = lax.dot_general(kk, v, (((0,), (0,)), ((), ())),
                           preferred_element_type=jnp.float32)

    @pl.when(i == 0)
    def _():
        ssqq_ref[...] = pssqq
        ssqk_ref[...] = pssqk
        kt1_ref[...] = pkt1
        ktv_ref[...] = pktv

    @pl.when(i > 0)
    def _():
        ssqq_ref[...] += pssqq
        ssqk_ref[...] += pssqk
        kt1_ref[...] += pkt1
        ktv_ref[...] += pktv


def _k1(x_pad, W_in, b_in, Wq, bq, Wk, bk, Wv, bv):
    nb = _NPAD // _B1
    row_spec = pl.BlockSpec((_B1, _H), lambda i: (i, 0))
    w_spec = pl.BlockSpec((_D, _H), lambda i: (0, 0))
    b_spec = pl.BlockSpec((1, _H), lambda i: (0, 0))
    return pl.pallas_call(
        _k1_body,
        grid=(nb,),
        in_specs=[row_spec, w_spec, b_spec, w_spec, b_spec, w_spec, b_spec,
                  w_spec, b_spec],
        out_specs=[row_spec, row_spec, row_spec,
                   pl.BlockSpec((1, 1), lambda i: (0, 0)),
                   pl.BlockSpec((1, 1), lambda i: (0, 0)),
                   pl.BlockSpec((1, _H), lambda i: (0, 0)),
                   pl.BlockSpec((_H, _H), lambda i: (0, 0))],
        out_shape=[jax.ShapeDtypeStruct((_NPAD, _H), jnp.float32),
                   jax.ShapeDtypeStruct((_NPAD, _H), jnp.float32),
                   jax.ShapeDtypeStruct((_NPAD, _H), jnp.float32),
                   jax.ShapeDtypeStruct((1, 1), jnp.float32),
                   jax.ShapeDtypeStruct((1, 1), jnp.float32),
                   jax.ShapeDtypeStruct((1, _H), jnp.float32),
                   jax.ShapeDtypeStruct((_H, _H), jnp.float32)],
    )(x_pad, W_in, b_in.reshape(1, _H), Wq, bq.reshape(1, _H),
      Wk, bk.reshape(1, _H), Wv, bv.reshape(1, _H))


def _k2_body(q_ref, v_ref, h_ref, deg_ref, ssqq_ref, ssqk_ref, kt1_ref,
             ktv_ref, Wg_ref, bg_ref,
             glog_ref, s_ref, hs_ref):
    qn = jnp.maximum(jnp.sqrt(ssqq_ref[...]), 1e-12)
    kn = jnp.maximum(jnp.sqrt(ssqk_ref[...]), 1e-12)
    sc = 1.0 / (qn * kn * jnp.float32(_N))
    q = q_ref[...]
    v = v_ref[...]
    an = 1.0 + jnp.sum(q * kt1_ref[...], axis=1, keepdims=True) * sc
    grep = (v + jnp.dot(q, ktv_ref[...],
                        preferred_element_type=jnp.float32) * sc)
    grep = grep / jnp.maximum(an, 1e-12)
    glog_ref[...] = jnp.dot(grep, Wg_ref[...],
                            preferred_element_type=jnp.float32) + bg_ref[...]
    deg = deg_ref[...]
    s = jnp.where(deg > 0, 1.0 / jnp.sqrt(jnp.maximum(deg, 1.0)), 0.0)
    s_ref[...] = s
    hs_ref[...] = s * h_ref[...]


def _k2(q, v, h, deg, ssqq, ssqk, kt1, ktv, Wg, bg):
    nb = _NPAD // _B1
    row_spec = pl.BlockSpec((_B1, _H), lambda i: (i, 0))
    return pl.pallas_call(
        _k2_body,
        grid=(nb,),
        in_specs=[row_spec, row_spec, row_spec,
                  pl.BlockSpec((_B1, 1), lambda i: (i, 0)),
                  pl.BlockSpec((1, 1), lambda i: (0, 0)),
                  pl.BlockSpec((1, 1), lambda i: (0, 0)),
                  pl.BlockSpec((1, _H), lambda i: (0, 0)),
                  pl.BlockSpec((_H, _H), lambda i: (0, 0)),
                  pl.BlockSpec((_H, _K), lambda i: (0, 0)),
                  pl.BlockSpec((1, _K), lambda i: (0, 0))],
        out_specs=[pl.BlockSpec((_B1, _K), lambda i: (i, 0)),
                   pl.BlockSpec((_B1, 1), lambda i: (i, 0)),
                   row_spec],
        out_shape=[jax.ShapeDtypeStruct((_NPAD, _K), jnp.float32),
                   jax.ShapeDtypeStruct((_NPAD, 1), jnp.float32),
                   jax.ShapeDtypeStruct((_NPAD, _H), jnp.float32)],
    )(q, v, h, deg, ssqq, ssqk, kt1, ktv, Wg, bg.reshape(1, _K))


def _k3_mid_body(h_ref, p_ref, s_ref, glog_ref, Wenv_ref, benv_ref,
                 Wcg_ref, Wch_ref, h_out_ref, hs_out_ref):
    h = h_ref[...]
    s = s_ref[...]
    p = p_ref[...]
    gcn = s * (p[0] + p[1])
    logits = jnp.dot(h, Wenv_ref[...],
                     preferred_element_type=jnp.float32) + benv_ref[...] + glog_ref[...]
    m = jnp.max(logits, axis=1, keepdims=True)
    e = jnp.exp(logits - m)
    w = e / jnp.sum(e, axis=1, keepdims=True)
    big = (jnp.dot(gcn, Wcg_ref[...], preferred_element_type=jnp.float32)
           + jnp.dot(h, Wch_ref[...], preferred_element_type=jnp.float32))
    acc = h
    for kk in range(_K):
        acc = acc + w[:, kk:kk + 1] * big[:, kk * _H:(kk + 1) * _H]
    hn = jnp.maximum(acc, 0.0)
    h_out_ref[...] = hn
    hs_out_ref[...] = s * hn


def _k3_last_body(h_ref, p_ref, s_ref, glog_ref, Wenv_ref, benv_ref,
                  Wcg_ref, Wch_ref, Wout_ref, bout_ref, y_ref):
    h = h_ref[...]
    s = s_ref[...]
    p = p_ref[...]
    gcn = s * (p[0] + p[1])
    logits = jnp.dot(h, Wenv_ref[...],
                     preferred_element_type=jnp.float32) + benv_ref[...] + glog_ref[...]
    m = jnp.max(logits, axis=1, keepdims=True)
    e = jnp.exp(logits - m)
    w = e / jnp.sum(e, axis=1, keepdims=True)
    big = (jnp.dot(gcn, Wcg_ref[...], preferred_element_type=jnp.float32)
           + jnp.dot(h, Wch_ref[...], preferred_element_type=jnp.float32))
    acc = h
    for kk in range(_K):
        acc = acc + w[:, kk:kk + 1] * big[:, kk * _H:(kk + 1) * _H]
    hn = jnp.maximum(acc, 0.0)
    y_ref[...] = jnp.dot(hn, Wout_ref[...],
                         preferred_element_type=jnp.float32) + bout_ref[...]


def _k3_specs():
    row_spec = pl.BlockSpec((_B3, _H), lambda i: (i, 0))
    return [row_spec,
            pl.BlockSpec((2, _B3, _H), lambda i: (0, i, 0)),
            pl.BlockSpec((_B3, 1), lambda i: (i, 0)),
            pl.BlockSpec((_B3, _K), lambda i: (i, 0)),
            pl.BlockSpec((_H, _K), lambda i: (0, 0)),
            pl.BlockSpec((1, _K), lambda i: (0, 0)),
            pl.BlockSpec((_H, _K * _H), lambda i: (0, 0)),
            pl.BlockSpec((_H, _K * _H), lambda i: (0, 0))]


def _k3_mid(h, p, s, glog, Wenv, benv, Wcg, Wch):
    nb = _NPAD // _B3
    row_spec = pl.BlockSpec((_B3, _H), lambda i: (i, 0))
    return pl.pallas_call(
        _k3_mid_body,
        grid=(nb,),
        in_specs=_k3_specs(),
        out_specs=[row_spec, row_spec],
        out_shape=[jax.ShapeDtypeStruct((_NPAD, _H), jnp.float32),
                   jax.ShapeDtypeStruct((_NPAD, _H), jnp.float32)],
    )(h, p, s, glog, Wenv, benv.reshape(1, _K), Wcg, Wch)


def _k3_last(h, p, s, glog, Wenv, benv, Wcg, Wch, W_out, b_out):
    nb = _NPAD // _B3
    return pl.pallas_call(
        _k3_last_body,
        grid=(nb,),
        in_specs=_k3_specs() + [pl.BlockSpec((_H, _C), lambda i: (0, 0)),
                                pl.BlockSpec((1, _C), lambda i: (0, 0))],
        out_specs=[pl.BlockSpec((_B3, _C), lambda i: (i, 0))],
        out_shape=[jax.ShapeDtypeStruct((_NPAD, _C), jnp.float32)],
    )(h, p, s, glog, Wenv, benv.reshape(1, _K), Wcg, Wch, W_out,
      b_out.reshape(1, _C))[0]


def kernel(x, edge_index, W_in, b_in, Wq, bq, Wk, bk, Wv, bv, Wg, bg,
           W_env_enc, b_env_enc, W_conv, W_out, b_out):
    x_pad = jnp.pad(x, ((0, _NPAD - _N), (0, 0)))
    row = edge_index[0]
    col = edge_index[1]
    npad_e = _EPAD - _E
    row_pad = jnp.concatenate(
        [row, jnp.zeros((npad_e,), jnp.int32)]).reshape(_EPAD // _CHUNK, _CHUNK)
    col_pad = jnp.concatenate(
        [col, jnp.full((npad_e,), _NPAD - 1, jnp.int32)]).reshape(
            _EPAD // _CHUNK, _CHUNK)
    zeros_acc = jnp.zeros((_NPAD, _H), jnp.float32)

    deg2 = _deg_partials(col_pad, jnp.zeros((_NPAD,), jnp.float32))
    deg = jnp.sum(deg2, axis=0).reshape(_NPAD, 1)

    h, q, v, ssqq, ssqk, kt1, ktv = _k1(x_pad, W_in, b_in, Wq, bq, Wk, bk,
                                        Wv, bv)
    glog, s, hs = _k2(q, v, h, deg, ssqq, ssqk, kt1, ktv, Wg, bg)

    Wcg0 = W_conv[0][:, :_H, :].transpose(1, 0, 2).reshape(_H, _K * _H)
    Wch0 = W_conv[0][:, _H:, :].transpose(1, 0, 2).reshape(_H, _K * _H)
    Wcg1 = W_conv[1][:, :_H, :].transpose(1, 0, 2).reshape(_H, _K * _H)
    Wch1 = W_conv[1][:, _H:, :].transpose(1, 0, 2).reshape(_H, _K * _H)

    p0 = _gcn_partials(hs, row_pad, col_pad, zeros_acc)
    h, hs = _k3_mid(h, p0, s, glog, W_env_enc[0], b_env_enc[0], Wcg0, Wch0)

    p1 = _gcn_partials(hs, row_pad, col_pad, zeros_acc)
    y = _k3_last(h, p1, s, glog, W_env_enc[1], b_env_enc[1], Wcg1, Wch1,
                 W_out, b_out)
    return y[:_N]


# speedup vs baseline: 1.0466x; 1.0316x over previous
import dataclasses
import functools

import jax
import jax.numpy as jnp
from jax import lax
from jax.experimental import pallas as pl
from jax.experimental.pallas import tpu as pltpu
from jax.experimental.pallas import tpu_sc as plsc

_N = 10000
_E = 320000
_D = 128
_H = 128
_C = 64
_K = 8

_NPAD = 10240
_NC = 2
_NS = 16
_CHUNK = 128
_CPW = 80
_EPW = _CHUNK * _CPW
_EPAD = _EPW * _NC * _NS
_IBLK = 16

_B1 = 1024
_B3 = 512


def _deg_partials(col_pad, zeros1):
    mesh = plsc.VectorSubcoreMesh(core_axis_name="c", subcore_axis_name="s")
    cp = pltpu.CompilerParams()
    if "needs_layout_passes" in pltpu.CompilerParams.__dataclass_fields__:
        cp = dataclasses.replace(cp, needs_layout_passes=False)

    @functools.partial(
        pl.kernel, mesh=mesh, compiler_params=cp,
        out_type=jax.ShapeDtypeStruct((_NC * _NS, _NPAD), jnp.float32),
        scratch_types=[
            pltpu.VMEM((_CPW, _CHUNK), jnp.int32),
            pltpu.VMEM((_NPAD,), jnp.float32),
        ],
    )
    def k(col_hbm, z1_hbm, out_hbm, cidx, hist):
        cid = lax.axis_index("c")
        sid = lax.axis_index("s")
        wid = cid * _NS + sid
        pltpu.sync_copy(z1_hbm, hist)
        pltpu.sync_copy(col_hbm.at[pl.ds(wid * _CPW, _CPW)], cidx)
        ones16 = jnp.full((16,), 1.0, jnp.float32)

        @pl.loop(0, _CPW)
        def _(c):
            @pl.loop(0, _CHUNK // 16)
            def _(j):
                idxv = cidx[c, pl.ds(j * 16, 16)]
                plsc.addupdate_scatter(hist, [idxv], ones16)

        pltpu.sync_copy(hist, out_hbm.at[wid])

    return k(col_pad, zeros1)


_CPW0 = 144
_CPW1 = 16


def _gcn_partials(hs, row_pad, col_pad, zeros_acc):
    mesh = plsc.VectorSubcoreMesh(core_axis_name="c", subcore_axis_name="s")

    @functools.partial(
        pl.kernel, mesh=mesh,
        out_type=jax.ShapeDtypeStruct((_NC, _NPAD, _H), jnp.float32),
        scratch_types=[
            pltpu.VMEM((_IBLK, _CHUNK), jnp.int32),
            pltpu.VMEM((_IBLK, _CHUNK), jnp.int32),
            pltpu.VMEM((_CHUNK, _H), jnp.float32),
            pltpu.VMEM((_CHUNK, _H), jnp.float32),
            pltpu.VMEM_SHARED((_NPAD, _H), jnp.float32),
            pltpu.SemaphoreType.DMA,
            pltpu.SemaphoreType.DMA,
        ],
    )
    def k(hs_hbm, row_hbm, col_hbm, z_hbm, out_hbm, ridx, cidx, rows0, rows1,
          acc, sem0, sem1):
        cid = lax.axis_index("c")
        sid = lax.axis_index("s")
        rows_per = _NPAD // _NS
        pltpu.sync_copy(z_hbm.at[pl.ds(sid * rows_per, rows_per)],
                        acc.at[pl.ds(sid * rows_per, rows_per)])
        my_cpw = jnp.where(cid == 0, _CPW0, _CPW1)
        base_c = jnp.where(cid == 0, sid * _CPW0,
                           _NS * _CPW0 + sid * _CPW1)
        plsc.subcore_barrier()

        @pl.loop(0, my_cpw, step=_IBLK)
        def _(b):
            pltpu.sync_copy(row_hbm.at[pl.ds(base_c + b, _IBLK)], ridx)
            pltpu.sync_copy(col_hbm.at[pl.ds(base_c + b, _IBLK)], cidx)
            pltpu.async_copy(hs_hbm.at[ridx.at[0]], rows0, sem0)

            @pl.loop(0, _IBLK, step=2)
            def _(c):
                pltpu.async_copy(hs_hbm.at[ridx.at[c + 1]], rows1, sem1)
                pltpu.make_async_copy(hs_hbm.at[ridx.at[c]], rows0,
                                      sem0).wait()
                pltpu.sync_copy(rows0, acc.at[cidx.at[c]], add=True)

                @pl.when(c + 2 < _IBLK)
                def _():
                    pltpu.async_copy(hs_hbm.at[ridx.at[c + 2]], rows0, sem0)

                pltpu.make_async_copy(hs_hbm.at[ridx.at[c + 1]], rows1,
                                      sem1).wait()
                pltpu.sync_copy(rows1, acc.at[cidx.at[c + 1]], add=True)

        plsc.subcore_barrier()
        pltpu.sync_copy(acc.at[pl.ds(sid * rows_per, rows_per)],
                        out_hbm.at[cid, pl.ds(sid * rows_per, rows_per)])

    return k(hs, row_pad, col_pad, zeros_acc)


def _k1_body(x_ref, Win_ref, bin_ref, Wq_ref, bq_ref, Wk_ref, bk_ref,
             Wv_ref, bv_ref,
             h_ref, q_ref, v_ref, ssqq_ref, ssqk_ref, kt1_ref, ktv_ref):
    i = pl.program_id(0)
    xb = x_ref[...]
    h = jnp.maximum(jnp.dot(xb, Win_ref[...],
                            preferred_element_type=jnp.float32) + bin_ref[...], 0.0)
    ridx = lax.broadcasted_iota(jnp.int32, (_B1, 1), 0) + i * _B1
    mask = ridx < _N
    h = jnp.where(mask, h, 0.0)
    q = jnp.where(mask, jnp.dot(h, Wq_ref[...],
                                preferred_element_type=jnp.float32) + bq_ref[...], 0.0)
    kk = jnp.where(mask, jnp.dot(h, Wk_ref[...],
                                 preferred_element_type=jnp.float32) + bk_ref[...], 0.0)
    v = jnp.where(mask, jnp.dot(h, Wv_ref[...],
                                preferred_element_type=jnp.float32) + bv_ref[...], 0.0)
    h_ref[...] = h
    q_ref[...] = q
    v_ref[...] = v
    pssqq = jnp.sum(q * q, axis=(0, 1), keepdims=True)
    pssqk = jnp.sum(kk * kk, axis=(0, 1), keepdims=True)
    pkt1 = jnp.sum(kk, axis=0, keepdims=True)
    pktv = lax.dot_general(kk, v, (((0,), (0,)), ((), ())),
                           preferred_element_type=jnp.float32)

    @pl.when(i == 0)
    def _():
        ssqq_ref[...] = pssqq
        ssqk_ref[...] = pssqk
        kt1_ref[...] = pkt1
        ktv_ref[...] = pktv

    @pl.when(i > 0)
    def _():
        ssqq_ref[...] += pssqq
        ssqk_ref[...] += pssqk
        kt1_ref[...] += pkt1
        ktv_ref[...] += pktv


def _k1(x_pad, W_in, b_in, Wq, bq, Wk, bk, Wv, bv):
    nb = _NPAD // _B1
    row_spec = pl.BlockSpec((_B1, _H), lambda i: (i, 0))
    w_spec = pl.BlockSpec((_D, _H), lambda i: (0, 0))
    b_spec = pl.BlockSpec((1, _H), lambda i: (0, 0))
    return pl.pallas_call(
        _k1_body,
        grid=(nb,),
        in_specs=[row_spec, w_spec, b_spec, w_spec, b_spec, w_spec, b_spec,
                  w_spec, b_spec],
        out_specs=[row_spec, row_spec, row_spec,
                   pl.BlockSpec((1, 1), lambda i: (0, 0)),
                   pl.BlockSpec((1, 1), lambda i: (0, 0)),
                   pl.BlockSpec((1, _H), lambda i: (0, 0)),
                   pl.BlockSpec((_H, _H), lambda i: (0, 0))],
        out_shape=[jax.ShapeDtypeStruct((_NPAD, _H), jnp.float32),
                   jax.ShapeDtypeStruct((_NPAD, _H), jnp.float32),
                   jax.ShapeDtypeStruct((_NPAD, _H), jnp.float32),
                   jax.ShapeDtypeStruct((1, 1), jnp.float32),
                   jax.ShapeDtypeStruct((1, 1), jnp.float32),
                   jax.ShapeDtypeStruct((1, _H), jnp.float32),
                   jax.ShapeDtypeStruct((_H, _H), jnp.float32)],
    )(x_pad, W_in, b_in.reshape(1, _H), Wq, bq.reshape(1, _H),
      Wk, bk.reshape(1, _H), Wv, bv.reshape(1, _H))


def _k2_body(q_ref, v_ref, h_ref, deg_ref, ssqq_ref, ssqk_ref, kt1_ref,
             ktv_ref, Wg_ref, bg_ref,
             glog_ref, s_ref, hs_ref):
    qn = jnp.maximum(jnp.sqrt(ssqq_ref[...]), 1e-12)
    kn = jnp.maximum(jnp.sqrt(ssqk_ref[...]), 1e-12)
    sc = 1.0 / (qn * kn * jnp.float32(_N))
    q = q_ref[...]
    v = v_ref[...]
    an = 1.0 + jnp.sum(q * kt1_ref[...], axis=1, keepdims=True) * sc
    grep = (v + jnp.dot(q, ktv_ref[...],
                        preferred_element_type=jnp.float32) * sc)
    grep = grep / jnp.maximum(an, 1e-12)
    glog_ref[...] = jnp.dot(grep, Wg_ref[...],
                            preferred_element_type=jnp.float32) + bg_ref[...]
    deg = deg_ref[...]
    s = jnp.where(deg > 0, 1.0 / jnp.sqrt(jnp.maximum(deg, 1.0)), 0.0)
    s_ref[...] = s
    hs_ref[...] = s * h_ref[...]


def _k2(q, v, h, deg, ssqq, ssqk, kt1, ktv, Wg, bg):
    nb = _NPAD // _B1
    row_spec = pl.BlockSpec((_B1, _H), lambda i: (i, 0))
    return pl.pallas_call(
        _k2_body,
        grid=(nb,),
        in_specs=[row_spec, row_spec, row_spec,
                  pl.BlockSpec((_B1, 1), lambda i: (i, 0)),
                  pl.BlockSpec((1, 1), lambda i: (0, 0)),
                  pl.BlockSpec((1, 1), lambda i: (0, 0)),
                  pl.BlockSpec((1, _H), lambda i: (0, 0)),
                  pl.BlockSpec((_H, _H), lambda i: (0, 0)),
                  pl.BlockSpec((_H, _K), lambda i: (0, 0)),
                  pl.BlockSpec((1, _K), lambda i: (0, 0))],
        out_specs=[pl.BlockSpec((_B1, _K), lambda i: (i, 0)),
                   pl.BlockSpec((_B1, 1), lambda i: (i, 0)),
                   row_spec],
        out_shape=[jax.ShapeDtypeStruct((_NPAD, _K), jnp.float32),
                   jax.ShapeDtypeStruct((_NPAD, 1), jnp.float32),
                   jax.ShapeDtypeStruct((_NPAD, _H), jnp.float32)],
    )(q, v, h, deg, ssqq, ssqk, kt1, ktv, Wg, bg.reshape(1, _K))


def _k3_mid_body(h_ref, p_ref, s_ref, glog_ref, Wenv_ref, benv_ref,
                 Wcg_ref, Wch_ref, h_out_ref, hs_out_ref):
    h = h_ref[...]
    s = s_ref[...]
    p = p_ref[...]
    gcn = s * (p[0] + p[1])
    logits = jnp.dot(h, Wenv_ref[...],
                     preferred_element_type=jnp.float32) + benv_ref[...] + glog_ref[...]
    m = jnp.max(logits, axis=1, keepdims=True)
    e = jnp.exp(logits - m)
    w = e / jnp.sum(e, axis=1, keepdims=True)
    big = (jnp.dot(gcn, Wcg_ref[...], preferred_element_type=jnp.float32)
           + jnp.dot(h, Wch_ref[...], preferred_element_type=jnp.float32))
    acc = h
    for kk in range(_K):
        acc = acc + w[:, kk:kk + 1] * big[:, kk * _H:(kk + 1) * _H]
    hn = jnp.maximum(acc, 0.0)
    h_out_ref[...] = hn
    hs_out_ref[...] = s * hn


def _k3_last_body(h_ref, p_ref, s_ref, glog_ref, Wenv_ref, benv_ref,
                  Wcg_ref, Wch_ref, Wout_ref, bout_ref, y_ref):
    h = h_ref[...]
    s = s_ref[...]
    p = p_ref[...]
    gcn = s * (p[0] + p[1])
    logits = jnp.dot(h, Wenv_ref[...],
                     preferred_element_type=jnp.float32) + benv_ref[...] + glog_ref[...]
    m = jnp.max(logits, axis=1, keepdims=True)
    e = jnp.exp(logits - m)
    w = e / jnp.sum(e, axis=1, keepdims=True)
    big = (jnp.dot(gcn, Wcg_ref[...], preferred_element_type=jnp.float32)
           + jnp.dot(h, Wch_ref[...], preferred_element_type=jnp.float32))
    acc = h
    for kk in range(_K):
        acc = acc + w[:, kk:kk + 1] * big[:, kk * _H:(kk + 1) * _H]
    hn = jnp.maximum(acc, 0.0)
    y_ref[...] = jnp.dot(hn, Wout_ref[...],
                         preferred_element_type=jnp.float32) + bout_ref[...]


def _k3_specs():
    row_spec = pl.BlockSpec((_B3, _H), lambda i: (i, 0))
    return [row_spec,
            pl.BlockSpec((2, _B3, _H), lambda i: (0, i, 0)),
            pl.BlockSpec((_B3, 1), lambda i: (i, 0)),
            pl.BlockSpec((_B3, _K), lambda i: (i, 0)),
            pl.BlockSpec((_H, _K), lambda i: (0, 0)),
            pl.BlockSpec((1, _K), lambda i: (0, 0)),
            pl.BlockSpec((_H, _K * _H), lambda i: (0, 0)),
            pl.BlockSpec((_H, _K * _H), lambda i: (0, 0))]


def _k3_mid(h, p, s, glog, Wenv, benv, Wcg, Wch):
    nb = _NPAD // _B3
    row_spec = pl.BlockSpec((_B3, _H), lambda i: (i, 0))
    return pl.pallas_call(
        _k3_mid_body,
        grid=(nb,),
        in_specs=_k3_specs(),
        out_specs=[row_spec, row_spec],
        out_shape=[jax.ShapeDtypeStruct((_NPAD, _H), jnp.float32),
                   jax.ShapeDtypeStruct((_NPAD, _H), jnp.float32)],
    )(h, p, s, glog, Wenv, benv.reshape(1, _K), Wcg, Wch)


def _k3_last(h, p, s, glog, Wenv, benv, Wcg, Wch, W_out, b_out):
    nb = _NPAD // _B3
    return pl.pallas_call(
        _k3_last_body,
        grid=(nb,),
        in_specs=_k3_specs() + [pl.BlockSpec((_H, _C), lambda i: (0, 0)),
                                pl.BlockSpec((1, _C), lambda i: (0, 0))],
        out_specs=[pl.BlockSpec((_B3, _C), lambda i: (i, 0))],
        out_shape=[jax.ShapeDtypeStruct((_NPAD, _C), jnp.float32)],
    )(h, p, s, glog, Wenv, benv.reshape(1, _K), Wcg, Wch, W_out,
      b_out.reshape(1, _C))[0]


def kernel(x, edge_index, W_in, b_in, Wq, bq, Wk, bk, Wv, bv, Wg, bg,
           W_env_enc, b_env_enc, W_conv, W_out, b_out):
    x_pad = jnp.pad(x, ((0, _NPAD - _N), (0, 0)))
    row = edge_index[0]
    col = edge_index[1]
    npad_e = _EPAD - _E
    row_pad = jnp.concatenate(
        [row, jnp.zeros((npad_e,), jnp.int32)]).reshape(_EPAD // _CHUNK, _CHUNK)
    col_pad = jnp.concatenate(
        [col, jnp.full((npad_e,), _NPAD - 1, jnp.int32)]).reshape(
            _EPAD // _CHUNK, _CHUNK)
    zeros_acc = jnp.zeros((_NPAD, _H), jnp.float32)

    deg2 = _deg_partials(col_pad, jnp.zeros((_NPAD,), jnp.float32))
    deg = jnp.sum(deg2, axis=0).reshape(_NPAD, 1)

    h, q, v, ssqq, ssqk, kt1, ktv = _k1(x_pad, W_in, b_in, Wq, bq, Wk, bk,
                                        Wv, bv)
    glog, s, hs = _k2(q, v, h, deg, ssqq, ssqk, kt1, ktv, Wg, bg)

    Wcg0 = W_conv[0][:, :_H, :].transpose(1, 0, 2).reshape(_H, _K * _H)
    Wch0 = W_conv[0][:, _H:, :].transpose(1, 0, 2).reshape(_H, _K * _H)
    Wcg1 = W_conv[1][:, :_H, :].transpose(1, 0, 2).reshape(_H, _K * _H)
    Wch1 = W_conv[1][:, _H:, :].transpose(1, 0, 2).reshape(_H, _K * _H)

    p0 = _gcn_partials(hs, row_pad, col_pad, zeros_acc)
    h, hs = _k3_mid(h, p0, s, glog, W_env_enc[0], b_env_enc[0], Wcg0, Wch0)

    p1 = _gcn_partials(hs, row_pad, col_pad, zeros_acc)
    y = _k3_last(h, p1, s, glog, W_env_enc[1], b_env_enc[1], Wcg1, Wch1,
                 W_out, b_out)
    return y[:_N]
